# baseline (speedup 1.0000x reference)
import functools

import jax
import jax.numpy as jnp
from jax import lax
from jax.experimental import pallas as pl
from jax.experimental.pallas import tpu as pltpu
from jax.experimental.pallas import tpu_sc as plsc

_NC = 2
_NS = 16
_NW = _NC * _NS
_CHUNK = 800


def _make_gather(B, V, D):
    b_per_w = B // _NW
    nchunks = b_per_w // _CHUNK
    npairs = nchunks // 2
    mesh = plsc.VectorSubcoreMesh(core_axis_name="c", subcore_axis_name="s")

    @functools.partial(
        pl.kernel,
        mesh=mesh,
        out_type=jax.ShapeDtypeStruct((B, 2 * D), jnp.float32),
        compiler_params=pltpu.CompilerParams(use_tc_tiling_on_sc=False),
        scratch_types=[
            pltpu.VMEM((_CHUNK,), jnp.int32),
            pltpu.VMEM((_CHUNK,), jnp.int32),
            pltpu.VMEM((_CHUNK, D), jnp.float32),
            pltpu.VMEM((_CHUNK, D), jnp.float32),
            pltpu.SemaphoreType.DMA,
            pltpu.SemaphoreType.DMA,
            pltpu.SemaphoreType.DMA,
        ],
    )
    def k(idx_hbm, table_hbm, out_hbm, idx_v0, idx_v1, rows_v0, rows_v1,
          isem, gsem, osem):
        wid = lax.axis_index("s") * _NC + lax.axis_index("c")
        base = wid * b_per_w
        idx_bufs = (idx_v0, idx_v1)
        row_bufs = (rows_v0, rows_v1)
        n = nchunks

        def idx_copy(c, s):
            return pltpu.make_async_copy(
                idx_hbm.at[pl.ds(base + c * _CHUNK, _CHUNK)],
                idx_bufs[s],
                isem,
            )

        def gather_copy(s):
            return pltpu.make_async_copy(
                table_hbm.at[idx_bufs[s]], row_bufs[s], gsem
            )

        def store_copy(c, s):
            return pltpu.make_async_copy(
                row_bufs[s],
                out_hbm.at[pl.ds(base + c * _CHUNK, _CHUNK), pl.ds(0, D)],
                osem,
            )

        idx_copy(0, 0).start()
        idx_copy(0, 0).wait()
        gather_copy(0).start()
        idx_copy(1, 1).start()

        def pair_body(g, carry):
            for sbuf in range(2):
                c = g * 2 + sbuf
                obuf = 1 - sbuf

                @pl.when(c < n - 1)
                def _():
                    @pl.when(c >= 1)
                    def _():
                        store_copy(c - 1, obuf).wait()

                    idx_copy(c + 1, obuf).wait()
                    gather_copy(obuf).start()

                gather_copy(sbuf).wait()
                store_copy(c, sbuf).start()

                @pl.when(c < n - 2)
                def _():
                    idx_copy(c + 2, sbuf).start()
            return carry

        lax.fori_loop(0, npairs, pair_body, 0)

        store_copy(n - 2, (n - 2) % 2).wait()
        store_copy(n - 1, (n - 1) % 2).wait()

    return k


def kernel(x, table):
    Bb, H = x.shape
    V, D = table.shape
    B = Bb * H
    idx_flat = x.reshape(B).astype(jnp.int32)
    tab_lin = lax.optimization_barrier(table.reshape(V * D))
    tab2 = tab_lin.reshape(V, D)
    wide = _make_gather(B, V, D)(idx_flat, tab2)
    return wide[:, :D].reshape(Bb, H, D)


# speedup vs baseline: 1.0018x; 1.0018x over previous
import functools

import jax
import jax.numpy as jnp
from jax import lax
from jax.experimental import pallas as pl
from jax.experimental.pallas import tpu as pltpu
from jax.experimental.pallas import tpu_sc as plsc

_NC = 2
_NS = 16
_NW = _NC * _NS
_CHUNK = 800


def _make_gather(B, V, D):
    b_per_w = B // _NW
    nchunks = b_per_w // _CHUNK
    npairs = nchunks // 2
    mesh = plsc.VectorSubcoreMesh(core_axis_name="c", subcore_axis_name="s")

    @functools.partial(
        pl.kernel,
        mesh=mesh,
        out_type=jax.ShapeDtypeStruct((B, 2 * D), jnp.float32),
        compiler_params=pltpu.CompilerParams(use_tc_tiling_on_sc=False),
        scratch_types=[
            pltpu.VMEM((_CHUNK,), jnp.int32),
            pltpu.VMEM((_CHUNK,), jnp.int32),
            pltpu.VMEM((_CHUNK, D), jnp.float32),
            pltpu.VMEM((_CHUNK, D), jnp.float32),
            pltpu.SemaphoreType.DMA,
            pltpu.SemaphoreType.DMA,
            pltpu.SemaphoreType.DMA,
        ],
    )
    def k(idx_hbm, table_hbm, out_hbm, idx_v0, idx_v1, rows_v0, rows_v1,
          isem, gsem, osem):
        wid = lax.axis_index("s") * _NC + lax.axis_index("c")
        base = wid * b_per_w
        idx_bufs = (idx_v0, idx_v1)
        row_bufs = (rows_v0, rows_v1)
        n = nchunks

        def idx_copy(c, s):
            return pltpu.make_async_copy(
                idx_hbm.at[pl.ds(base + c * _CHUNK, _CHUNK)],
                idx_bufs[s],
                isem,
            )

        def gather_copy(s):
            return pltpu.make_async_copy(
                table_hbm.at[idx_bufs[s]], row_bufs[s], gsem
            )

        def store_copy(c, s):
            return pltpu.make_async_copy(
                row_bufs[s],
                out_hbm.at[pl.ds(base + c * _CHUNK, _CHUNK), pl.ds(0, D)],
                osem,
            )

        idx_copy(0, 0).start()
        idx_copy(0, 0).wait()
        gather_copy(0).start()
        idx_copy(1, 1).start()

        def pair_body(g, carry):
            for sbuf in range(2):
                c = g * 2 + sbuf
                obuf = 1 - sbuf

                @pl.when(c < n - 1)
                def _():
                    @pl.when(c >= 1)
                    def _():
                        store_copy(c - 1, obuf).wait()

                    idx_copy(c + 1, obuf).wait()
                    gather_copy(obuf).start()

                gather_copy(sbuf).wait()
                store_copy(c, sbuf).start()

                @pl.when(c < n - 2)
                def _():
                    idx_copy(c + 2, sbuf).start()
            return carry

        lax.fori_loop(0, npairs, pair_body, 0)

        store_copy(n - 2, (n - 2) % 2).wait()
        store_copy(n - 1, (n - 1) % 2).wait()

    return k


def kernel(x, table):
    Bb, H = x.shape
    V, D = table.shape
    B = Bb * H
    idx_flat = x.reshape(B).astype(jnp.int32)
    wide = _make_gather(B, V, D)(idx_flat, table)
    return wide[:, :D].reshape(Bb, H, D)
